# baseline (speedup 1.0000x reference)
import functools

import jax
import jax.numpy as jnp
from jax import lax
from jax.experimental import pallas as pl
from jax.experimental.pallas import tpu as pltpu
from jax.experimental.pallas import tpu_sc as plsc

B = 16384
F = 128
L = 4

_NC, _NS = 1, 16
_NW = _NC * _NS
_CHUNK = B // _NW
_GROUPS = _CHUNK // 128


P = 1000


def _sc_gather_body(idx_hbm, mu_hbm, out_hbm, idx_v, mu_v, mug_v, sem):
    wid = lax.axis_index("s") * _NC + lax.axis_index("c")
    base = wid * _CHUNK
    c_idx = pltpu.async_copy(idx_hbm.at[pl.ds(base, _CHUNK)], idx_v, sem)
    c_tab = pltpu.async_copy(mu_hbm, mu_v.at[pl.ds(0, P)], sem)
    c_idx.wait()
    c_tab.wait()
    def _step(k, carry):
        off = k * 16
        ids = idx_v[pl.ds(off, 16)]
        mug_v[pl.ds(off, 16)] = plsc.load_gather(mu_v, [ids])
        return carry

    lax.fori_loop(0, _CHUNK // 16, _step, 0)
    pltpu.sync_copy(mug_v, out_hbm.at[0, pl.ds(base, _CHUNK)])


@functools.lru_cache(maxsize=1)
def _sc_gather():
    return functools.partial(
        pl.kernel,
        out_type=jax.ShapeDtypeStruct((1, B), jnp.float32),
        mesh=plsc.VectorSubcoreMesh(core_axis_name="c", subcore_axis_name="s",
                                    num_cores=1),
        compiler_params=pltpu.CompilerParams(needs_layout_passes=False,
                                             skip_device_barrier=True),
        scratch_types=[
            pltpu.VMEM((_CHUNK,), jnp.int32),
            pltpu.VMEM((1024,), jnp.float32),
            pltpu.VMEM((_CHUNK,), jnp.float32),
            pltpu.SemaphoreType.DMA,
        ],
    )(_sc_gather_body)


_RB = 2048


def _tc_matvec_body(bu_ref, bi_ref, sc_ref, u_ref, i_ref, out_ref):
    dn = (((1,), (1,)), ((), ()))
    base = lax.dot_general(bu_ref[...], u_ref[...], dn,
                           preferred_element_type=jnp.float32)
    base += lax.dot_general(bi_ref[...], i_ref[...], dn,
                            preferred_element_type=jnp.float32)
    out_ref[...] = base + sc_ref[0]


_tc_matvec = pl.pallas_call(
    _tc_matvec_body,
    grid=(B // _RB,),
    in_specs=[
        pl.BlockSpec((1, F), lambda i: (0, 0)),
        pl.BlockSpec((1, F), lambda i: (0, 0)),
        pl.BlockSpec(memory_space=pltpu.SMEM),
        pl.BlockSpec((_RB, F), lambda i: (i, 0)),
        pl.BlockSpec((_RB, F), lambda i: (i, 0)),
    ],
    out_specs=pl.BlockSpec((1, _RB), lambda i: (0, i)),
    out_shape=jax.ShapeDtypeStruct((1, B), jnp.float32),
)


def _tc_combine_body(sc_ref, base_ref, mug_ref, eps_ref, out_ref):
    sigma = jnp.exp(sc_ref[0])
    out_ref[...] = (base_ref[...]
                    + jnp.exp(mug_ref[...] + sigma * eps_ref[...])).reshape(-1)


_CB = 8192
_NB = B // _CB

_tc_combine = pl.pallas_call(
    _tc_combine_body,
    grid=(L, _NB),
    in_specs=[
        pl.BlockSpec(memory_space=pltpu.SMEM),
        pl.BlockSpec((1, _CB), lambda l, i: (0, i)),
        pl.BlockSpec((1, _CB), lambda l, i: (0, i)),
        pl.BlockSpec((1, 1, _CB), lambda l, i: (l, 0, i)),
    ],
    out_specs=pl.BlockSpec((_CB,), lambda l, i: (l * _NB + i,)),
    out_shape=jax.ShapeDtypeStruct((L * B,), jnp.float32),
)


def kernel(users, items, items_pop_idx, beta_user, beta_item, intercept,
           pop_bias_mu, pop_bias_log_sigma, eps, L_arg):
    idx = items_pop_idx.astype(jnp.int32)
    mug = _sc_gather()(idx, pop_bias_mu)
    lsig = pop_bias_log_sigma.astype(jnp.float32).reshape(1)
    base = _tc_matvec(beta_user.reshape(1, F), beta_item.reshape(1, F),
                      intercept, users, items)
    return _tc_combine(lsig, base, mug, eps.reshape(L, 1, B))


# speedup vs baseline: 1.2216x; 1.2216x over previous
import functools

import jax
import jax.numpy as jnp
from jax import lax
from jax.experimental import pallas as pl
from jax.experimental.pallas import tpu as pltpu
from jax.experimental.pallas import tpu_sc as plsc

B = 16384
F = 128
L = 4

_NC, _NS = 1, 16
_NW = _NC * _NS
_CHUNK = B // _NW
_GROUPS = _CHUNK // 128


P = 1000


def _sc_gather_body(idx_hbm, mu_hbm, out_hbm, idx_v, mu_v, mug_v, sem):
    wid = lax.axis_index("s") * _NC + lax.axis_index("c")
    base = wid * _CHUNK
    c_idx = pltpu.async_copy(idx_hbm.at[pl.ds(base, _CHUNK)], idx_v, sem)
    c_tab = pltpu.async_copy(mu_hbm, mu_v.at[pl.ds(0, P)], sem)
    c_idx.wait()
    c_tab.wait()
    def _step(k, carry):
        off = k * 16
        ids = idx_v[pl.ds(off, 16)]
        mug_v[pl.ds(off, 16)] = plsc.load_gather(mu_v, [ids])
        return carry

    lax.fori_loop(0, _CHUNK // 16, _step, 0)
    pltpu.sync_copy(mug_v, out_hbm.at[0, pl.ds(base, _CHUNK)])


@functools.lru_cache(maxsize=1)
def _sc_gather():
    return functools.partial(
        pl.kernel,
        out_type=jax.ShapeDtypeStruct((1, B), jnp.float32),
        mesh=plsc.VectorSubcoreMesh(core_axis_name="c", subcore_axis_name="s",
                                    num_cores=1),
        compiler_params=pltpu.CompilerParams(needs_layout_passes=False,
                                             skip_device_barrier=True),
        scratch_types=[
            pltpu.VMEM((_CHUNK,), jnp.int32),
            pltpu.VMEM((1024,), jnp.float32),
            pltpu.VMEM((_CHUNK,), jnp.float32),
            pltpu.SemaphoreType.DMA,
        ],
    )(_sc_gather_body)


_RB = 4096


def _tc_matvec_body(bu_ref, bi_ref, sc_ref, u_ref, i_ref, out_ref):
    dn = (((1,), (1,)), ((), ()))
    base = lax.dot_general(bu_ref[...], u_ref[...], dn,
                           preferred_element_type=jnp.float32)
    base += lax.dot_general(bi_ref[...], i_ref[...], dn,
                            preferred_element_type=jnp.float32)
    out_ref[...] = base + sc_ref[0]


_tc_matvec = pl.pallas_call(
    _tc_matvec_body,
    grid=(B // _RB,),
    in_specs=[
        pl.BlockSpec((1, F), lambda i: (0, 0)),
        pl.BlockSpec((1, F), lambda i: (0, 0)),
        pl.BlockSpec(memory_space=pltpu.SMEM),
        pl.BlockSpec((_RB, F), lambda i: (i, 0)),
        pl.BlockSpec((_RB, F), lambda i: (i, 0)),
    ],
    out_specs=pl.BlockSpec((1, _RB), lambda i: (0, i)),
    out_shape=jax.ShapeDtypeStruct((1, B), jnp.float32),
)


def _tc_combine_body(sc_ref, base_ref, mug_ref, eps_ref, out_ref):
    sigma = jnp.exp(sc_ref[0])
    out_ref[...] = (base_ref[...]
                    + jnp.exp(mug_ref[...] + sigma * eps_ref[...]))


_CB = 8192

_tc_combine = pl.pallas_call(
    _tc_combine_body,
    grid=(B // _CB,),
    in_specs=[
        pl.BlockSpec(memory_space=pltpu.SMEM),
        pl.BlockSpec((1, _CB), lambda i: (0, i)),
        pl.BlockSpec((1, _CB), lambda i: (0, i)),
        pl.BlockSpec((L, _CB), lambda i: (0, i)),
    ],
    out_specs=pl.BlockSpec((L, _CB), lambda i: (0, i)),
    out_shape=jax.ShapeDtypeStruct((L, B), jnp.float32),
)


def kernel(users, items, items_pop_idx, beta_user, beta_item, intercept,
           pop_bias_mu, pop_bias_log_sigma, eps, L_arg):
    idx = items_pop_idx.astype(jnp.int32)
    mug = _sc_gather()(idx, pop_bias_mu)
    lsig = pop_bias_log_sigma.astype(jnp.float32).reshape(1)
    base = _tc_matvec(beta_user.reshape(1, F), beta_item.reshape(1, F),
                      intercept, users, items)
    out = _tc_combine(lsig, base, mug, eps)
    return jnp.reshape(out, (-1,))
